# baseline (speedup 1.0000x reference)
import functools
import math

import numpy as np
import jax
from jax import lax
import jax.numpy as jnp
from jax.experimental import pallas as pl
from jax.experimental.pallas import tpu as pltpu
from jax.experimental.pallas import tpu_sc as plsc

_V = 32000
_N = 2048
_S = float(np.float32(0.1 / (_V - 2)))
_C = 0.9
_CONST_PER_ROW = (_V - 2) * _S * math.log(_S) + _C * math.log(_C)

_N_SC = 2048
_N_TC = _N - _N_SC


_BR = 128
_BC = 32000
_NRB = _N_TC // _BR
_NCB = _V // _BC


def _tc_body(t_ref, x_ref, o_ref, acc_ref):
    i = pl.program_id(0)
    j = pl.program_id(1)

    @pl.when((i == 0) & (j == 0))
    def _():
        acc_ref[0] = 0.0
        acc_ref[1] = 0.0
        acc_ref[2] = 0.0

    xb = x_ref[...]
    t = t_ref[0, 0, :]
    col = jax.lax.broadcasted_iota(jnp.int32, (_BR, _BC), 1) + j * _BC
    xz = jnp.where(col == 0, 0.0, xb)
    rowp = jnp.sum(xz, axis=1)
    rowp = jnp.where(t == 0, 0.0, rowp)
    acc_ref[0] += jnp.sum(rowp)
    match = (col == t[:, None]) & (t[:, None] != 0)
    acc_ref[1] += jnp.sum(jnp.where(match, xb, 0.0))

    @pl.when(j == 0)
    def _():
        acc_ref[2] += jnp.sum((t != 0).astype(jnp.float32))

    @pl.when((i == _NRB - 1) & (j == _NCB - 1))
    def _():
        o_ref[0] = acc_ref[0]
        o_ref[1] = acc_ref[1]
        o_ref[2] = acc_ref[2]


def _tc_part(x, t3):
    return pl.pallas_call(
        _tc_body,
        grid=(_NRB, _NCB),
        in_specs=[
            pl.BlockSpec((1, 1, _BR), lambda i, j: (i, 0, 0)),
            pl.BlockSpec((_BR, _BC), lambda i, j: (i, j)),
        ],
        out_specs=pl.BlockSpec(memory_space=pltpu.SMEM),
        out_shape=jax.ShapeDtypeStruct((3,), jnp.float32),
        scratch_shapes=[pltpu.SMEM((3,), jnp.float32)],
    )(t3, x)


_NC = 2
_NS = 16
_NW = _NC * _NS
_RPW = _N_SC // _NW
_CHUNKS = _V // (16 * 8)


def _sc_body(x_hbm, t_hbm, out_hbm, t_v, buf_a, buf_b, stage_v, sem):
    wid = lax.axis_index("s") * _NC + lax.axis_index("c")
    base = _N_TC + wid * _RPW

    pltpu.async_copy(x_hbm.at[base], buf_a, sem)
    pltpu.sync_copy(t_hbm.at[pl.ds(base, _RPW)], t_v)

    lane0 = lax.iota(jnp.int32, 16) == 0
    zeros16 = jnp.zeros((16,), jnp.float32)
    zidx = jnp.zeros((16,), jnp.int32)

    dense_acc = zeros16
    x0_acc = zeros16
    g_acc = zeros16
    cnt_acc = zeros16

    for r in range(_RPW):
        buf = buf_a if r % 2 == 0 else buf_b
        if r + 1 < _RPW:
            nbuf = buf_b if r % 2 == 0 else buf_a
            pltpu.async_copy(x_hbm.at[base + r + 1], nbuf, sem)
        pltpu.make_async_copy(x_hbm.at[base + r], buf, sem).wait()

        def chunk_body(i, accs, buf=buf):
            new = []
            for k in range(8):
                new.append(accs[k] + buf[pl.ds(i * 128 + k * 16, 16)])
            return tuple(new)

        accs = lax.fori_loop(0, _CHUNKS, chunk_body,
                             tuple(zeros16 for _ in range(8)))
        row_vec = (((accs[0] + accs[1]) + (accs[2] + accs[3]))
                   + ((accs[4] + accs[5]) + (accs[6] + accs[7])))

        t_rep = plsc.load_gather(t_v, [jnp.full((16,), r, jnp.int32)])
        mask = t_rep != 0
        m0 = mask & lane0
        g_rep = plsc.load_gather(buf, [t_rep])
        x0_rep = plsc.load_gather(buf, [zidx])

        dense_acc = dense_acc + jnp.where(mask, row_vec, 0.0)
        x0_acc = x0_acc + jnp.where(m0, x0_rep, 0.0)
        g_acc = g_acc + jnp.where(m0, g_rep, 0.0)
        cnt_acc = cnt_acc + jnp.where(m0, 1.0, 0.0)

    stage_v[0] = dense_acc
    stage_v[1] = x0_acc
    stage_v[2] = g_acc
    stage_v[3] = cnt_acc
    pltpu.sync_copy(stage_v, out_hbm.at[wid])


_sc_part = functools.partial(
    pl.kernel,
    mesh=plsc.VectorSubcoreMesh(core_axis_name="c", subcore_axis_name="s"),
    out_type=jax.ShapeDtypeStruct((_NW, 4, 16), jnp.float32),
    scratch_types=[
        pltpu.VMEM((_RPW,), jnp.int32),
        pltpu.VMEM((_V,), jnp.float32),
        pltpu.VMEM((_V,), jnp.float32),
        pltpu.VMEM((4, 16), jnp.float32),
        pltpu.SemaphoreType.DMA,
    ],
    compiler_params=pltpu.CompilerParams(needs_layout_passes=False),
)(_sc_body)


def kernel(x, target):
    t32 = target.astype(jnp.int32)
    parts = _sc_part(x, t32)
    if _N_TC:
        tc = _tc_part(x, t32.reshape(_N // _BR, 1, _BR))
    else:
        tc = jnp.zeros((3,), jnp.float32)
    dense = tc[0] + (jnp.sum(parts[:, 0, :]) - jnp.sum(parts[:, 1, :]))
    g = tc[1] + jnp.sum(parts[:, 2, :])
    k = tc[2] + jnp.sum(parts[:, 3, :])
    return (k * _CONST_PER_ROW - (_S * dense + (_C - _S) * g)) / _N


# speedup vs baseline: 1.2696x; 1.2696x over previous
import functools
import math

import numpy as np
import jax
from jax import lax
import jax.numpy as jnp
from jax.experimental import pallas as pl
from jax.experimental.pallas import tpu as pltpu
from jax.experimental.pallas import tpu_sc as plsc

_V = 32000
_N = 2048
_S = float(np.float32(0.1 / (_V - 2)))
_C = 0.9
_CONST_PER_ROW = (_V - 2) * _S * math.log(_S) + _C * math.log(_C)

_N_SC = 1024
_N_TC = _N - _N_SC


_BR = 128
_BC = 32000
_NRB = _N_TC // _BR
_NCB = _V // _BC


def _tc_body(t_ref, x_ref, o_ref, acc_ref):
    i = pl.program_id(0)
    j = pl.program_id(1)

    @pl.when((i == 0) & (j == 0))
    def _():
        acc_ref[0] = 0.0
        acc_ref[1] = 0.0
        acc_ref[2] = 0.0

    xb = x_ref[...]
    t = t_ref[0, 0, :]
    col = jax.lax.broadcasted_iota(jnp.int32, (_BR, _BC), 1) + j * _BC
    xz = jnp.where(col == 0, 0.0, xb)
    rowp = jnp.sum(xz, axis=1)
    rowp = jnp.where(t == 0, 0.0, rowp)
    acc_ref[0] += jnp.sum(rowp)
    match = (col == t[:, None]) & (t[:, None] != 0)
    acc_ref[1] += jnp.sum(jnp.where(match, xb, 0.0))

    @pl.when(j == 0)
    def _():
        acc_ref[2] += jnp.sum((t != 0).astype(jnp.float32))

    @pl.when((i == _NRB - 1) & (j == _NCB - 1))
    def _():
        o_ref[0] = acc_ref[0]
        o_ref[1] = acc_ref[1]
        o_ref[2] = acc_ref[2]


def _tc_part(x, t3):
    return pl.pallas_call(
        _tc_body,
        grid=(_NRB, _NCB),
        in_specs=[
            pl.BlockSpec((1, 1, _BR), lambda i, j: (i, 0, 0)),
            pl.BlockSpec((_BR, _BC), lambda i, j: (i, j)),
        ],
        out_specs=pl.BlockSpec(memory_space=pltpu.SMEM),
        out_shape=jax.ShapeDtypeStruct((3,), jnp.float32),
        scratch_shapes=[pltpu.SMEM((3,), jnp.float32)],
    )(t3, x)


_NC = 2
_NS = 16
_NW = _NC * _NS
_RPW = _N_SC // _NW
_CHUNKS = _V // (16 * 8)


def _sc_body(x_hbm, t_hbm, out_hbm, t_v, buf_a, buf_b, stage_v, sem):
    wid = lax.axis_index("s") * _NC + lax.axis_index("c")
    base = _N_TC + wid * _RPW

    pltpu.async_copy(x_hbm.at[base], buf_a, sem)
    pltpu.sync_copy(t_hbm.at[pl.ds(base, _RPW)], t_v)

    lane0 = lax.iota(jnp.int32, 16) == 0
    zeros16 = jnp.zeros((16,), jnp.float32)
    zidx = jnp.zeros((16,), jnp.int32)

    dense_acc = zeros16
    x0_acc = zeros16
    g_acc = zeros16
    cnt_acc = zeros16

    for r in range(_RPW):
        buf = buf_a if r % 2 == 0 else buf_b
        if r + 1 < _RPW:
            nbuf = buf_b if r % 2 == 0 else buf_a
            pltpu.async_copy(x_hbm.at[base + r + 1], nbuf, sem)
        pltpu.make_async_copy(x_hbm.at[base + r], buf, sem).wait()

        def chunk_body(i, accs, buf=buf):
            new = []
            for k in range(8):
                new.append(accs[k] + buf[pl.ds(i * 128 + k * 16, 16)])
            return tuple(new)

        accs = lax.fori_loop(0, _CHUNKS, chunk_body,
                             tuple(zeros16 for _ in range(8)))
        row_vec = (((accs[0] + accs[1]) + (accs[2] + accs[3]))
                   + ((accs[4] + accs[5]) + (accs[6] + accs[7])))

        t_rep = plsc.load_gather(t_v, [jnp.full((16,), r, jnp.int32)])
        mask = t_rep != 0
        m0 = mask & lane0
        g_rep = plsc.load_gather(buf, [t_rep])
        x0_rep = plsc.load_gather(buf, [zidx])

        dense_acc = dense_acc + jnp.where(mask, row_vec, 0.0)
        x0_acc = x0_acc + jnp.where(m0, x0_rep, 0.0)
        g_acc = g_acc + jnp.where(m0, g_rep, 0.0)
        cnt_acc = cnt_acc + jnp.where(m0, 1.0, 0.0)

    stage_v[0] = dense_acc
    stage_v[1] = x0_acc
    stage_v[2] = g_acc
    stage_v[3] = cnt_acc
    pltpu.sync_copy(stage_v, out_hbm.at[wid])


_sc_part = functools.partial(
    pl.kernel,
    mesh=plsc.VectorSubcoreMesh(core_axis_name="c", subcore_axis_name="s"),
    out_type=jax.ShapeDtypeStruct((_NW, 4, 16), jnp.float32),
    scratch_types=[
        pltpu.VMEM((_RPW,), jnp.int32),
        pltpu.VMEM((_V,), jnp.float32),
        pltpu.VMEM((_V,), jnp.float32),
        pltpu.VMEM((4, 16), jnp.float32),
        pltpu.SemaphoreType.DMA,
    ],
    compiler_params=pltpu.CompilerParams(needs_layout_passes=False),
)(_sc_body)


def kernel(x, target):
    t32 = target.astype(jnp.int32)
    parts = _sc_part(x, t32)
    if _N_TC:
        tc = _tc_part(x, t32.reshape(_N // _BR, 1, _BR))
    else:
        tc = jnp.zeros((3,), jnp.float32)
    dense = tc[0] + (jnp.sum(parts[:, 0, :]) - jnp.sum(parts[:, 1, :]))
    g = tc[1] + jnp.sum(parts[:, 2, :])
    k = tc[2] + jnp.sum(parts[:, 3, :])
    return (k * _CONST_PER_ROW - (_S * dense + (_C - _S) * g)) / _N
